# Initial kernel scaffold; baseline (speedup 1.0000x reference)
import jax
import jax.numpy as jnp
from jax.experimental import pallas as pl


def kernel(x, W_enc, b_enc, W_dec, b_dec):
    raise NotImplementedError("write your pallas kernel here")


# speedup vs baseline: 8.1027x; 8.1027x over previous
import functools

import jax
import jax.numpy as jnp
from jax import lax
from jax.experimental import pallas as pl
from jax.experimental.pallas import tpu as pltpu
from jax.experimental.pallas import tpu_sc as plsc

B = 2048
D = 2048
N = 16384
K = 64

NC = 2
NS = 16
L = 16
NW = NC * NS
RPW = B // NW

NGROUP = 64
GCH = 32
SV = 16
NSTRIPE = D // (SV * L)

_BIG = 1 << 20


def _enc_body(x_ref, w_ref, bdec_ref, benc_ref, o_ref):
    xm = x_ref[...] - bdec_ref[...]
    acc = lax.dot_general(xm, w_ref[...], (((1,), (1,)), ((), ())),
                          preferred_element_type=jnp.float32)
    o_ref[...] = jnp.maximum(acc + benc_ref[...], 0.0)


def _encode(x, W_enc, b_dec2, b_enc2):
    NBLK = 512
    return pl.pallas_call(
        _enc_body,
        grid=(N // NBLK,),
        in_specs=[
            pl.BlockSpec((B, D), lambda i: (0, 0)),
            pl.BlockSpec((NBLK, D), lambda i: (i, 0)),
            pl.BlockSpec((1, D), lambda i: (0, 0)),
            pl.BlockSpec((1, NBLK), lambda i: (0, i)),
        ],
        out_specs=pl.BlockSpec((B, NBLK), lambda i: (0, i)),
        out_shape=jax.ShapeDtypeStruct((B, N), jnp.float32),
    )(x, W_enc, b_dec2, b_enc2)


def _sc_body(pre_hbm, wdec_hbm, vals_hbm, cols_hbm, out_hbm,
             row_v, l1_v, vals_v, cols_v, gbuf_v, acc_v, sem):
    wid = lax.axis_index("s") * NC + lax.axis_index("c")
    row0 = wid * RPW
    iota = lax.iota(jnp.int32, L)
    lane0 = iota == 0

    def _sets(ref, idx_scalars, val):
        idxs = [jnp.full((L,), i, jnp.int32) for i in idx_scalars]
        plsc.store_scatter(ref, idxs, jnp.full((L,), val), mask=lane0)

    def row_body(r, _):
        row = row0 + r
        pltpu.sync_copy(pre_hbm.at[row], row_v)

        def build(j, l2):
            base = j * 256
            acc = row_v[pl.ds(base, L)]
            for i in range(1, 16):
                acc = jnp.maximum(acc, row_v[pl.ds(base + i * L, L)])
            l1_v[pl.ds(j * L, L)] = acc
            return jnp.maximum(l2, acc)

        l2 = lax.fori_loop(0, NGROUP, build,
                           jnp.full((L,), -1.0, jnp.float32))

        def ext(t, l2):
            m = jnp.max(l2)
            lane = jnp.min(jnp.where(l2 == m, iota, L))
            best = jnp.zeros((L,), jnp.int32)
            for q in range(NGROUP // L):
                idx = (iota + q * L) * L + lane
                v = plsc.load_gather(l1_v, [idx])
                best = jnp.maximum(
                    best, jnp.where(v == m, _BIG - (iota + q * L), 0))
            j = _BIG - jnp.max(best)
            cidx = j * 256 + iota * L + lane
            cand = plsc.load_gather(row_v, [cidx])
            i_ = _BIG - jnp.max(jnp.where(cand == m, _BIG - iota, 0))
            col = j * 256 + i_ * L + lane
            _sets(vals_v, [t], m)
            _sets(cols_v, [t // GCH, t % GCH], col)
            _sets(row_v, [col], jnp.float32(-1.0))
            newcand = jnp.where(iota == i_, jnp.float32(-1.0), cand)
            g1 = jnp.max(newcand)
            _sets(l1_v, [j * L + lane], g1)
            best2 = jnp.full((L,), -1.0, jnp.float32)
            for q in range(NGROUP // L):
                idx = (iota + q * L) * L + lane
                best2 = jnp.maximum(best2, plsc.load_gather(l1_v, [idx]))
            g2 = jnp.max(best2)
            return jnp.where(iota == lane, g2, l2)

        lax.fori_loop(0, K, ext, l2)

        for chunk in range(K // GCH):
            cp = pltpu.async_copy(wdec_hbm.at[cols_v.at[chunk]], gbuf_v, sem)
            cp.wait()
            for stripe in range(NSTRIPE):
                base = stripe * SV * L
                if chunk == 0:
                    regs = tuple(jnp.zeros((L,), jnp.float32)
                                 for _ in range(SV))
                else:
                    regs = tuple(acc_v[pl.ds(base + i * L, L)]
                                 for i in range(SV))

                def kb(k, regs, chunk=chunk, base=base):
                    a = plsc.load_gather(
                        vals_v, [jnp.full((L,), chunk * GCH + k, jnp.int32)])
                    return tuple(
                        regs[i] + a * gbuf_v[k, pl.ds(base + i * L, L)]
                        for i in range(SV))

                regs = lax.fori_loop(0, GCH, kb, regs)
                for i in range(SV):
                    acc_v[pl.ds(base + i * L, L)] = regs[i]

        pltpu.sync_copy(vals_v, vals_hbm.at[row])
        pltpu.sync_copy(cols_v, cols_hbm.at[row])
        pltpu.sync_copy(acc_v, out_hbm.at[row])
        return 0

    lax.fori_loop(0, RPW, row_body, 0)


def _sc_topk_decode(pre, W_dec):
    mesh = plsc.VectorSubcoreMesh(core_axis_name="c", subcore_axis_name="s")
    fn = functools.partial(
        pl.kernel,
        out_type=[
            jax.ShapeDtypeStruct((B, K), jnp.float32),
            jax.ShapeDtypeStruct((B, K // GCH, GCH), jnp.int32),
            jax.ShapeDtypeStruct((B, D), jnp.float32),
        ],
        mesh=mesh,
        scratch_types=[
            pltpu.VMEM((N,), jnp.float32),
            pltpu.VMEM((NGROUP * L,), jnp.float32),
            pltpu.VMEM((K,), jnp.float32),
            pltpu.VMEM((K // GCH, GCH), jnp.int32),
            pltpu.VMEM((GCH, D), jnp.float32),
            pltpu.VMEM((D,), jnp.float32),
            pltpu.SemaphoreType.DMA,
        ],
        compiler_params=pltpu.CompilerParams(needs_layout_passes=False),
    )(_sc_body)
    return fn(pre, W_dec)


def _fin_body(x_ref, p_ref, bdec_ref, sae_ref, fvu_ref, colsum_ref, s_ref):
    i = pl.program_id(0)

    @pl.when(i == 0)
    def _():
        colsum_ref[...] = jnp.zeros_like(colsum_ref)
        s_ref[0] = 0.0
        s_ref[1] = 0.0

    x = x_ref[...]
    sae = p_ref[...] + bdec_ref[...]
    sae_ref[...] = sae
    e = x - sae
    colsum_ref[...] += jnp.sum(x, axis=0, keepdims=True)
    s_ref[0] += jnp.sum(x * x)
    s_ref[1] += jnp.sum(e * e)
    nb = pl.num_programs(0)

    @pl.when(i == nb - 1)
    def _():
        cs = colsum_ref[...]
        tv = s_ref[0] - jnp.sum(cs * cs) / B
        fvu_ref[...] = jnp.full((1, 1), s_ref[1] / tv, jnp.float32)


def _finalize(x, partial, b_dec2):
    RB = 256
    return pl.pallas_call(
        _fin_body,
        grid=(B // RB,),
        in_specs=[
            pl.BlockSpec((RB, D), lambda i: (i, 0)),
            pl.BlockSpec((RB, D), lambda i: (i, 0)),
            pl.BlockSpec((1, D), lambda i: (0, 0)),
        ],
        out_specs=[
            pl.BlockSpec((RB, D), lambda i: (i, 0)),
            pl.BlockSpec((1, 1), lambda i: (0, 0)),
        ],
        out_shape=[
            jax.ShapeDtypeStruct((B, D), jnp.float32),
            jax.ShapeDtypeStruct((1, 1), jnp.float32),
        ],
        scratch_shapes=[
            pltpu.VMEM((1, D), jnp.float32),
            pltpu.SMEM((2,), jnp.float32),
        ],
    )(x, partial, b_dec2)


def kernel(x, W_enc, b_enc, W_dec, b_dec):
    b_dec2 = b_dec.reshape(1, D)
    b_enc2 = b_enc.reshape(1, N)
    pre = _encode(x, W_enc, b_dec2, b_enc2)
    top_acts, cols3, partial = _sc_topk_decode(pre, W_dec)
    top_indices = cols3.reshape(B, K)
    sae_out, fvu2 = _finalize(x, partial, b_dec2)
    fvu = fvu2[0, 0]
    zero = jnp.zeros((), x.dtype)
    return (sae_out, top_acts, top_indices, fvu, zero, zero)


# speedup vs baseline: 10.9553x; 1.3521x over previous
import functools

import jax
import jax.numpy as jnp
from jax import lax
from jax.experimental import pallas as pl
from jax.experimental.pallas import tpu as pltpu
from jax.experimental.pallas import tpu_sc as plsc

B = 2048
D = 2048
N = 16384
K = 64

NC = 2
NS = 16
L = 16
NW = NC * NS
RPW = B // NW

NGROUP = 64
GCH = 16
NCHUNK = K // GCH
SV = 16
NSTRIPE = D // (SV * L)

_BIG = 1 << 20


def _enc_body(x_ref, w_ref, bdec_ref, benc_ref, o_ref):
    xm = x_ref[...] - bdec_ref[...]
    acc = lax.dot_general(xm, w_ref[...], (((1,), (1,)), ((), ())),
                          preferred_element_type=jnp.float32)
    o_ref[...] = jnp.maximum(acc + benc_ref[...], 0.0)


def _encode(x, W_enc, b_dec2, b_enc2):
    NBLK = 512
    return pl.pallas_call(
        _enc_body,
        grid=(N // NBLK,),
        in_specs=[
            pl.BlockSpec((B, D), lambda i: (0, 0)),
            pl.BlockSpec((NBLK, D), lambda i: (i, 0)),
            pl.BlockSpec((1, D), lambda i: (0, 0)),
            pl.BlockSpec((1, NBLK), lambda i: (0, i)),
        ],
        out_specs=pl.BlockSpec((B, NBLK), lambda i: (0, i)),
        out_shape=jax.ShapeDtypeStruct((B, N), jnp.float32),
    )(x, W_enc, b_dec2, b_enc2)


def _sc_body(pre_hbm, wdec_hbm, vals_hbm, cols_hbm, out_hbm,
             rb0, rb1, l1_v, vals0, vals1, cols0, cols1, gA, gB, acc_v,
             sem_r0, sem_r1, sem_gA, sem_gB):
    wid = lax.axis_index("s") * NC + lax.axis_index("c")
    row0 = wid * RPW
    iota = lax.iota(jnp.int32, L)
    lane0 = iota == 0

    def _sets(ref, idx_scalars, val):
        idxs = [jnp.full((L,), i, jnp.int32) for i in idx_scalars]
        plsc.store_scatter(ref, idxs, jnp.full((L,), val), mask=lane0)

    def _extract(rb, vals_r, cols_r):
        def build(j, l2):
            base = j * 256
            acc = rb[pl.ds(base, L)]
            for i in range(1, 16):
                acc = jnp.maximum(acc, rb[pl.ds(base + i * L, L)])
            l1_v[pl.ds(j * L, L)] = acc
            return jnp.maximum(l2, acc)

        l2 = lax.fori_loop(0, NGROUP, build,
                           jnp.full((L,), -1.0, jnp.float32))

        def ext(t, l2):
            m = jnp.max(l2)
            lane = jnp.min(jnp.where(l2 == m, iota, L))
            best = jnp.zeros((L,), jnp.int32)
            for q in range(NGROUP // L):
                idx = (iota + q * L) * L + lane
                v = plsc.load_gather(l1_v, [idx])
                best = jnp.maximum(
                    best, jnp.where(v == m, _BIG - (iota + q * L), 0))
            j = _BIG - jnp.max(best)
            cidx = j * 256 + iota * L + lane
            cand = plsc.load_gather(rb, [cidx])
            i_ = _BIG - jnp.max(jnp.where(cand == m, _BIG - iota, 0))
            col = j * 256 + i_ * L + lane
            _sets(vals_r, [t], m)
            _sets(cols_r, [t // GCH, t % GCH], col)
            _sets(rb, [col], jnp.float32(-1.0))
            newcand = jnp.where(iota == i_, jnp.float32(-1.0), cand)
            _sets(l1_v, [j * L + lane], jnp.max(newcand))
            best2 = jnp.full((L,), -1.0, jnp.float32)
            for q in range(NGROUP // L):
                idx = (iota + q * L) * L + lane
                best2 = jnp.maximum(best2, plsc.load_gather(l1_v, [idx]))
            return jnp.where(iota == lane, jnp.max(best2), l2)

        lax.fori_loop(0, K, ext, l2)

    def _fire(cols_r, c, gb, sem):
        pltpu.async_copy(wdec_hbm.at[cols_r.at[c]], gb, sem)

    def _accum(gb, vals_r, c):
        def sbody(s, _):
            base = s * SV * L
            if c == 0:
                regs = tuple(jnp.zeros((L,), jnp.float32) for _ in range(SV))
            else:
                regs = tuple(acc_v[pl.ds(base + i * L, L)] for i in range(SV))

            def kb(k, regs):
                a = plsc.load_gather(
                    vals_r, [jnp.full((L,), c * GCH + k, jnp.int32)])
                return tuple(
                    regs[i] + a * gb[k, pl.ds(base + i * L, L)]
                    for i in range(SV))

            regs = lax.fori_loop(0, GCH, kb, regs)
            for i in range(SV):
                acc_v[pl.ds(base + i * L, L)] = regs[i]
            return 0

        lax.fori_loop(0, NSTRIPE, sbody, 0)

    def _decode(vals_r, cols_r, row):
        for c in range(NCHUNK):
            gb, sem = (gA, sem_gA) if c % 2 == 0 else (gB, sem_gB)
            pltpu.make_async_copy(wdec_hbm.at[cols_r.at[c]], gb, sem).wait()
            _accum(gb, vals_r, c)
            if c + 2 < NCHUNK:
                _fire(cols_r, c + 2, gb, sem)
        pltpu.sync_copy(acc_v, out_hbm.at[row])
        pltpu.sync_copy(vals_r, vals_hbm.at[row])
        pltpu.sync_copy(cols_r, cols_hbm.at[row])

    pltpu.sync_copy(pre_hbm.at[row0], rb0)
    pltpu.async_copy(pre_hbm.at[row0 + 1], rb1, sem_r1)
    _extract(rb0, vals0, cols0)

    def pair_body(i, _):
        r = row0 + 2 * i

        @pl.when(2 * i + 2 < RPW)
        def _():
            pltpu.async_copy(pre_hbm.at[r + 2], rb0, sem_r0)

        _fire(cols0, 0, gA, sem_gA)
        _fire(cols0, 1, gB, sem_gB)
        pltpu.make_async_copy(pre_hbm.at[r + 1], rb1, sem_r1).wait()
        _extract(rb1, vals1, cols1)
        _decode(vals0, cols0, r)

        @pl.when(2 * i + 3 < RPW)
        def _():
            pltpu.async_copy(pre_hbm.at[r + 3], rb1, sem_r1)

        _fire(cols1, 0, gA, sem_gA)
        _fire(cols1, 1, gB, sem_gB)

        @pl.when(2 * i + 2 < RPW)
        def _():
            pltpu.make_async_copy(pre_hbm.at[r + 2], rb0, sem_r0).wait()
            _extract(rb0, vals0, cols0)

        _decode(vals1, cols1, r + 1)
        return 0

    lax.fori_loop(0, RPW // 2, pair_body, 0)


def _sc_topk_decode(pre, W_dec):
    mesh = plsc.VectorSubcoreMesh(core_axis_name="c", subcore_axis_name="s")
    fn = functools.partial(
        pl.kernel,
        out_type=[
            jax.ShapeDtypeStruct((B, K), jnp.float32),
            jax.ShapeDtypeStruct((B, NCHUNK, GCH), jnp.int32),
            jax.ShapeDtypeStruct((B, D), jnp.float32),
        ],
        mesh=mesh,
        scratch_types=[
            pltpu.VMEM((N,), jnp.float32),
            pltpu.VMEM((N,), jnp.float32),
            pltpu.VMEM((NGROUP * L,), jnp.float32),
            pltpu.VMEM((K,), jnp.float32),
            pltpu.VMEM((K,), jnp.float32),
            pltpu.VMEM((NCHUNK, GCH), jnp.int32),
            pltpu.VMEM((NCHUNK, GCH), jnp.int32),
            pltpu.VMEM((GCH, D), jnp.float32),
            pltpu.VMEM((GCH, D), jnp.float32),
            pltpu.VMEM((D,), jnp.float32),
            pltpu.SemaphoreType.DMA,
            pltpu.SemaphoreType.DMA,
            pltpu.SemaphoreType.DMA,
            pltpu.SemaphoreType.DMA,
        ],
        compiler_params=pltpu.CompilerParams(needs_layout_passes=False),
    )(_sc_body)
    return fn(pre, W_dec)


def _fin_body(x_ref, p_ref, bdec_ref, sae_ref, fvu_ref, colsum_ref, s_ref):
    i = pl.program_id(0)

    @pl.when(i == 0)
    def _():
        colsum_ref[...] = jnp.zeros_like(colsum_ref)
        s_ref[0] = 0.0
        s_ref[1] = 0.0

    x = x_ref[...]
    sae = p_ref[...] + bdec_ref[...]
    sae_ref[...] = sae
    e = x - sae
    colsum_ref[...] += jnp.sum(x, axis=0, keepdims=True)
    s_ref[0] += jnp.sum(x * x)
    s_ref[1] += jnp.sum(e * e)
    nb = pl.num_programs(0)

    @pl.when(i == nb - 1)
    def _():
        cs = colsum_ref[...]
        tv = s_ref[0] - jnp.sum(cs * cs) / B
        fvu_ref[...] = jnp.full((1, 1), s_ref[1] / tv, jnp.float32)


def _finalize(x, partial, b_dec2):
    RB = 256
    return pl.pallas_call(
        _fin_body,
        grid=(B // RB,),
        in_specs=[
            pl.BlockSpec((RB, D), lambda i: (i, 0)),
            pl.BlockSpec((RB, D), lambda i: (i, 0)),
            pl.BlockSpec((1, D), lambda i: (0, 0)),
        ],
        out_specs=[
            pl.BlockSpec((RB, D), lambda i: (i, 0)),
            pl.BlockSpec((1, 1), lambda i: (0, 0)),
        ],
        out_shape=[
            jax.ShapeDtypeStruct((B, D), jnp.float32),
            jax.ShapeDtypeStruct((1, 1), jnp.float32),
        ],
        scratch_shapes=[
            pltpu.VMEM((1, D), jnp.float32),
            pltpu.SMEM((2,), jnp.float32),
        ],
    )(x, partial, b_dec2)


def kernel(x, W_enc, b_enc, W_dec, b_dec):
    b_dec2 = b_dec.reshape(1, D)
    b_enc2 = b_enc.reshape(1, N)
    pre = _encode(x, W_enc, b_dec2, b_enc2)
    top_acts, cols3, partial = _sc_topk_decode(pre, W_dec)
    top_indices = cols3.reshape(B, K)
    sae_out, fvu2 = _finalize(x, partial, b_dec2)
    fvu = fvu2[0, 0]
    zero = jnp.zeros((), x.dtype)
    return (sae_out, top_acts, top_indices, fvu, zero, zero)
